# baseline (device time: 80928 ns/iter reference)
import jax
import jax.numpy as jnp
from jax import lax
from jax.experimental import pallas as pl
from jax.experimental.pallas import tpu as pltpu

N_DEV = 8
E_LOC = 4
N_EXP = 32
T = 1024
D = 512
H = 1024
CAP = 64
CHUNK = E_LOC * CAP
G = N_EXP * CAP
P_SLOTS = 4


def kernel(x, router_W, route_idx, expert_W, shared_W):
    def body(x_ref, rw_ref, idx_ref, ew_ref, sw_ref, out_ref,
             gbuf, rxg, rbuf, rxr, ew_scr, spbuf,
             ag_send, ag_recv, rs_send, rs_recv):
        my = lax.axis_index("i")

        bar = pltpu.get_barrier_semaphore()
        for off in range(1, N_DEV):
            pl.semaphore_signal(bar, inc=1, device_id=((my + off) % N_DEV,),
                                device_id_type=pl.DeviceIdType.MESH)
        pl.semaphore_wait(bar, N_DEV - 1)

        scores = jnp.dot(x_ref[...], rw_ref[...],
                         preferred_element_type=jnp.float32)
        smax = jnp.max(scores, axis=-1, keepdims=True)
        exs = jnp.exp(scores - smax)
        probs = exs / jnp.sum(exs, axis=-1, keepdims=True)
        route = idx_ref[...]
        onehot = (lax.broadcasted_iota(jnp.int32, (T, N_EXP), 1)
                  == route).astype(jnp.float32)
        p = jnp.sum(onehot * probs, axis=-1, keepdims=True)

        ltri = (lax.broadcasted_iota(jnp.int32, (T, T), 0)
                >= lax.broadcasted_iota(jnp.int32, (T, T), 1)
                ).astype(jnp.bfloat16)
        cum = jnp.dot(ltri, onehot.astype(jnp.bfloat16),
                      preferred_element_type=jnp.float32)
        rank = (jnp.sum(onehot * cum, axis=-1, keepdims=True)
                .astype(jnp.int32) - 1)
        valid = rank < CAP
        slot_id = route * CAP + rank

        sel = ((lax.broadcasted_iota(jnp.int32, (T, G), 1) == slot_id)
               & valid).astype(jnp.bfloat16)
        spbuf[...] = sel * p.astype(jnp.bfloat16)

        gbuf[...] = lax.dot_general(
            sel, x_ref[...].astype(jnp.bfloat16),
            (((0,), (0,)), ((), ())),
            preferred_element_type=jnp.float32).astype(jnp.bfloat16)

        ag = []
        for off in range(1, N_DEV):
            dst = (my + off) % N_DEV
            r = pltpu.make_async_remote_copy(
                src_ref=gbuf.at[pl.ds(dst * CHUNK, CHUNK)],
                dst_ref=rxg.at[N_DEV - off],
                send_sem=ag_send.at[off],
                recv_sem=ag_recv.at[N_DEV - off],
                device_id=(dst,),
                device_id_type=pl.DeviceIdType.MESH,
            )
            r.start()
            ag.append(r)

        rxg[0] = gbuf[pl.ds(my * CHUNK, CHUNK), :]
        ew_scr[...] = ew_ref[...].astype(jnp.bfloat16)

        def expert_out(slot):
            blk = rxg[slot]
            parts = [
                jnp.dot(blk[el * CAP:(el + 1) * CAP, :], ew_scr[el],
                        preferred_element_type=jnp.float32)
                for el in range(E_LOC)
            ]
            return jnp.concatenate(parts, axis=0)

        out_ref[...] = jnp.dot(
            x_ref[...].astype(jnp.bfloat16), sw_ref[...].astype(jnp.bfloat16),
            preferred_element_type=jnp.float32)
        rxr[pl.ds(my * CHUNK, CHUNK), :] = expert_out(0).astype(jnp.bfloat16)

        rs = []
        for off in range(1, N_DEV):
            recv = pltpu.make_async_remote_copy(
                src_ref=gbuf.at[pl.ds(0, CHUNK)], dst_ref=rxg.at[off],
                send_sem=ag_send.at[off], recv_sem=ag_recv.at[off],
                device_id=((my + off) % N_DEV,),
                device_id_type=pl.DeviceIdType.MESH,
            )
            recv.wait_recv()
            slot = (off - 1) % P_SLOTS
            if off > P_SLOTS:
                rs[off - 1 - P_SLOTS].wait_send()
            rbuf[slot] = expert_out(off).astype(jnp.bfloat16)
            r = pltpu.make_async_remote_copy(
                src_ref=rbuf.at[slot],
                dst_ref=rxr.at[pl.ds(my * CHUNK, CHUNK)],
                send_sem=rs_send.at[off],
                recv_sem=rs_recv.at[N_DEV - off],
                device_id=((my + off) % N_DEV,),
                device_id_type=pl.DeviceIdType.MESH,
            )
            r.start()
            rs.append(r)

        for off in range(1, N_DEV):
            recv = pltpu.make_async_remote_copy(
                src_ref=rbuf.at[0], dst_ref=rxr.at[pl.ds(0, CHUNK)],
                send_sem=rs_send.at[off], recv_sem=rs_recv.at[off],
                device_id=((my + off) % N_DEV,),
                device_id_type=pl.DeviceIdType.MESH,
            )
            recv.wait_recv()
        out_ref[...] += jnp.dot(spbuf[...], rxr[...],
                                preferred_element_type=jnp.float32)

        for r in ag:
            r.wait_send()
        for r in rs[max(0, len(rs) - P_SLOTS):]:
            r.wait_send()

    return pl.pallas_call(
        body,
        out_shape=jax.ShapeDtypeStruct((T, H), jnp.float32),
        in_specs=[pl.BlockSpec(memory_space=pltpu.VMEM)] * 5,
        out_specs=pl.BlockSpec(memory_space=pltpu.VMEM),
        scratch_shapes=[
            pltpu.VMEM((G, D), jnp.bfloat16),
            pltpu.VMEM((N_DEV, CHUNK, D), jnp.bfloat16),
            pltpu.VMEM((P_SLOTS, CHUNK, H), jnp.bfloat16),
            pltpu.VMEM((G, H), jnp.bfloat16),
            pltpu.VMEM((E_LOC, D, H), jnp.bfloat16),
            pltpu.VMEM((T, G), jnp.bfloat16),
            pltpu.SemaphoreType.DMA((N_DEV,)),
            pltpu.SemaphoreType.DMA((N_DEV,)),
            pltpu.SemaphoreType.DMA((N_DEV,)),
            pltpu.SemaphoreType.DMA((N_DEV,)),
        ],
        compiler_params=pltpu.CompilerParams(
            collective_id=0, vmem_limit_bytes=62 * 1024 * 1024),
    )(x, router_W, route_idx, expert_W, shared_W)


# device time: 73715 ns/iter; 1.0978x vs baseline; 1.0978x over previous
import jax
import jax.numpy as jnp
from jax import lax
from jax.experimental import pallas as pl
from jax.experimental.pallas import tpu as pltpu

N_DEV = 8
E_LOC = 4
N_EXP = 32
T = 1024
D = 512
H = 1024
CAP = 64
CHUNK = E_LOC * CAP
G = N_EXP * CAP
P_SLOTS = 4


def kernel(x, router_W, route_idx, expert_W, shared_W):
    def body(x_ref, rw_ref, idx_ref, ew_ref, sw_ref, out_ref,
             gbuf, rxg, rbuf, rxr, ew_scr, spbuf,
             ag_send, ag_recv, rs_send, rs_recv):
        my = lax.axis_index("i")

        bar = pltpu.get_barrier_semaphore()
        for off in range(1, N_DEV):
            pl.semaphore_signal(bar, inc=1, device_id=((my + off) % N_DEV,),
                                device_id_type=pl.DeviceIdType.MESH)

        scores = jnp.dot(x_ref[...], rw_ref[...],
                         preferred_element_type=jnp.float32)
        smax = jnp.max(scores, axis=-1, keepdims=True)
        exs = jnp.exp(scores - smax)
        probs = exs / jnp.sum(exs, axis=-1, keepdims=True)
        route = idx_ref[...]
        onehot = (lax.broadcasted_iota(jnp.int32, (T, N_EXP), 1)
                  == route).astype(jnp.float32)
        p = jnp.sum(onehot * probs, axis=-1, keepdims=True)

        ltri = (lax.broadcasted_iota(jnp.int32, (T, T), 0)
                >= lax.broadcasted_iota(jnp.int32, (T, T), 1)
                ).astype(jnp.bfloat16)
        cum = jnp.dot(ltri, onehot.astype(jnp.bfloat16),
                      preferred_element_type=jnp.float32)
        rank = (jnp.sum(onehot * cum, axis=-1, keepdims=True)
                .astype(jnp.int32) - 1)
        valid = rank < CAP
        slot_id = route * CAP + rank

        xb = x_ref[...].astype(jnp.bfloat16)
        pb = p.astype(jnp.bfloat16)

        def sel_block(dst):
            c0 = dst * CHUNK
            iota = lax.broadcasted_iota(jnp.int32, (T, CHUNK), 1)
            return ((iota + c0 == slot_id) & valid).astype(jnp.bfloat16)

        ag = []
        for off in range(1, N_DEV):
            dst = (my + off) % N_DEV
            selb = sel_block(dst)
            gblk = lax.dot_general(
                selb, xb, (((0,), (0,)), ((), ())),
                preferred_element_type=jnp.float32)
            gbuf[pl.ds(dst * CHUNK, CHUNK), :] = gblk.astype(jnp.bfloat16)
            spbuf[:, pl.ds(dst * CHUNK, CHUNK)] = selb * pb
            if off == 1:
                pl.semaphore_wait(bar, N_DEV - 1)
            r = pltpu.make_async_remote_copy(
                src_ref=gbuf.at[pl.ds(dst * CHUNK, CHUNK)],
                dst_ref=rxg.at[N_DEV - off],
                send_sem=ag_send.at[off],
                recv_sem=ag_recv.at[N_DEV - off],
                device_id=(dst,),
                device_id_type=pl.DeviceIdType.MESH,
            )
            r.start()
            ag.append(r)

        selb = sel_block(my)
        rxg[0] = lax.dot_general(
            selb, xb, (((0,), (0,)), ((), ())),
            preferred_element_type=jnp.float32).astype(jnp.bfloat16)
        spbuf[:, pl.ds(my * CHUNK, CHUNK)] = selb * pb
        ew_scr[...] = ew_ref[...].astype(jnp.bfloat16)

        def expert_out(slot):
            blk = rxg[slot]
            parts = [
                jnp.dot(blk[el * CAP:(el + 1) * CAP, :], ew_scr[el],
                        preferred_element_type=jnp.float32)
                for el in range(E_LOC)
            ]
            return jnp.concatenate(parts, axis=0)

        out_ref[...] = jnp.dot(
            x_ref[...].astype(jnp.bfloat16), sw_ref[...].astype(jnp.bfloat16),
            preferred_element_type=jnp.float32) + jnp.dot(
            spbuf[:, pl.ds(my * CHUNK, CHUNK)],
            expert_out(0).astype(jnp.bfloat16),
            preferred_element_type=jnp.float32)

        rs = []
        for off in range(1, N_DEV):
            recv = pltpu.make_async_remote_copy(
                src_ref=gbuf.at[pl.ds(0, CHUNK)], dst_ref=rxg.at[off],
                send_sem=ag_send.at[off], recv_sem=ag_recv.at[off],
                device_id=((my + off) % N_DEV,),
                device_id_type=pl.DeviceIdType.MESH,
            )
            recv.wait_recv()
            slot = (off - 1) % P_SLOTS
            if off > P_SLOTS:
                rs[off - 1 - P_SLOTS].wait_send()
            rbuf[slot] = expert_out(off).astype(jnp.bfloat16)
            r = pltpu.make_async_remote_copy(
                src_ref=rbuf.at[slot],
                dst_ref=rxr.at[N_DEV - 1 - off],
                send_sem=rs_send.at[off],
                recv_sem=rs_recv.at[N_DEV - off],
                device_id=((my + off) % N_DEV,),
                device_id_type=pl.DeviceIdType.MESH,
            )
            r.start()
            rs.append(r)

        for off in range(1, N_DEV):
            dst = (my + off) % N_DEV
            recv = pltpu.make_async_remote_copy(
                src_ref=rbuf.at[0], dst_ref=rxr.at[off - 1],
                send_sem=rs_send.at[off], recv_sem=rs_recv.at[off],
                device_id=(dst,),
                device_id_type=pl.DeviceIdType.MESH,
            )
            recv.wait_recv()
            out_ref[...] += jnp.dot(
                spbuf[:, pl.ds(dst * CHUNK, CHUNK)], rxr[off - 1],
                preferred_element_type=jnp.float32)

        for r in ag:
            r.wait_send()
        for r in rs[max(0, len(rs) - P_SLOTS):]:
            r.wait_send()

    return pl.pallas_call(
        body,
        out_shape=jax.ShapeDtypeStruct((T, H), jnp.float32),
        in_specs=[pl.BlockSpec(memory_space=pltpu.VMEM)] * 5,
        out_specs=pl.BlockSpec(memory_space=pltpu.VMEM),
        scratch_shapes=[
            pltpu.VMEM((G, D), jnp.bfloat16),
            pltpu.VMEM((N_DEV, CHUNK, D), jnp.bfloat16),
            pltpu.VMEM((P_SLOTS, CHUNK, H), jnp.bfloat16),
            pltpu.VMEM((N_DEV - 1, CHUNK, H), jnp.bfloat16),
            pltpu.VMEM((E_LOC, D, H), jnp.bfloat16),
            pltpu.VMEM((T, G), jnp.bfloat16),
            pltpu.SemaphoreType.DMA((N_DEV,)),
            pltpu.SemaphoreType.DMA((N_DEV,)),
            pltpu.SemaphoreType.DMA((N_DEV,)),
            pltpu.SemaphoreType.DMA((N_DEV,)),
        ],
        compiler_params=pltpu.CompilerParams(
            collective_id=0, vmem_limit_bytes=62 * 1024 * 1024),
    )(x, router_W, route_idx, expert_W, shared_W)
